# Initial kernel scaffold; baseline (speedup 1.0000x reference)
import jax
import jax.numpy as jnp
from jax.experimental import pallas as pl


def kernel(dense, sparse, W_dense, b_dense, W_sparse, b_sparse, bias):
    raise NotImplementedError("write your pallas kernel here")


# speedup vs baseline: 4.3187x; 4.3187x over previous
import functools

import jax
import jax.numpy as jnp
from jax.experimental import pallas as pl

N = 16384
D_DENSE = 2048
D_SPARSE = 1024
E = 64
TOP_K = 8
BN = 512


def _body(d_ref, s_ref, wd_ref, bd_ref, ws_ref, bs_ref, b_ref,
          logits_ref, w_ref, m_ref, cd_ref, cs_ref):
    cd = jnp.dot(d_ref[...], wd_ref[...],
                 preferred_element_type=jnp.float32) + bd_ref[...]
    cs = jnp.dot(s_ref[...], ws_ref[...],
                 preferred_element_type=jnp.float32) + bs_ref[...]
    logits = b_ref[...] + cd + cs
    cd_ref[...] = cd
    cs_ref[...] = cs
    logits_ref[...] = logits

    col = jax.lax.broadcasted_iota(jnp.int32, logits.shape, 1)
    cur = logits
    mask = jnp.zeros(logits.shape, dtype=jnp.bool_)
    row_max = None
    for it in range(TOP_K):
        mx = jnp.max(cur, axis=-1, keepdims=True)
        if it == 0:
            row_max = mx
        cand = jnp.where(cur == mx, col, E)
        sel = col == jnp.min(cand, axis=-1, keepdims=True)
        mask = jnp.logical_or(mask, sel)
        cur = jnp.where(sel, -jnp.inf, cur)
    m_ref[...] = mask.astype(jnp.float32)

    e = jnp.where(mask, jnp.exp(logits - row_max), 0.0)
    w_ref[...] = e / jnp.sum(e, axis=-1, keepdims=True)


@jax.jit
def _router(dense, sparse, W_dense, b_dense, W_sparse, b_sparse, bias):
    grid = (N // BN,)
    out_shape = [jax.ShapeDtypeStruct((N, E), jnp.float32)] * 5
    row_spec = pl.BlockSpec((BN, E), lambda i: (i, 0))
    full = lambda shape: pl.BlockSpec(shape, lambda i: (0, 0))
    return pl.pallas_call(
        _body,
        grid=grid,
        in_specs=[
            pl.BlockSpec((BN, D_DENSE), lambda i: (i, 0)),
            pl.BlockSpec((BN, D_SPARSE), lambda i: (i, 0)),
            full((D_DENSE, E)),
            full((1, E)),
            full((D_SPARSE, E)),
            full((1, E)),
            full((1, E)),
        ],
        out_specs=[row_spec] * 5,
        out_shape=out_shape,
    )(dense, sparse, W_dense, b_dense.reshape(1, E),
      W_sparse, b_sparse.reshape(1, E), bias.reshape(1, E))


def kernel(dense, sparse, W_dense, b_dense, W_sparse, b_sparse, bias):
    logits, weights, topk_mask, c_dense, c_sparse = _router(
        dense, sparse, W_dense, b_dense, W_sparse, b_sparse, bias)
    return (logits, weights, topk_mask, c_dense, c_sparse)
